# baseline (speedup 1.0000x reference)
import jax
import jax.numpy as jnp
from jax import lax
from jax.experimental import pallas as pl
from jax.experimental.pallas import tpu as pltpu

B = 1024
C = 512
D = 64
NUM_CLASSES = 10
KAPPA_N = 10.0
CB = 16


def _egaussp_kernel(xT_ref, mu3_ref, sinv_ref, n_ref, labels_ref,
                    scores_ref, preds_ref, clusters_ref, gamma_ref):
    xT = xT_ref[:]
    ones = jnp.ones((CB, 1, D), jnp.float32)

    def step(i, carry):
        mu_b = mu3_ref[pl.ds(i * CB, CB), :, :]
        s_b = sinv_ref[pl.ds(i * CB, CB), :, :]
        diffT = xT[None, :, :] - mu_b
        tmpT = lax.dot_general(
            s_b, diffT, (((1,), (1,)), ((0,), (0,))),
            preferred_element_type=jnp.float32)
        d2 = lax.dot_general(
            ones, tmpT * diffT, (((2,), (1,)), ((0,), (0,))),
            preferred_element_type=jnp.float32)[:, 0, :]
        g = jnp.exp(-0.5 * d2)
        mask = n_ref[pl.ds(i * CB, CB), :] >= KAPPA_N
        g = jnp.where(mask, g, 0.0)
        gamma_ref[pl.ds(i * CB, CB), :] = g
        return carry

    lax.fori_loop(0, C // CB, step, 0, unroll=2)

    G = gamma_ref[:]
    denom = jnp.sum(G, axis=0) + 1e-12
    raw = lax.dot_general(
        G, labels_ref[:], (((0,), (0,)), ((), ())),
        preferred_element_type=jnp.float32)
    scores = raw / denom[:, None]
    scores_ref[:] = scores

    it = lax.broadcasted_iota(jnp.int32, scores.shape, 1)
    mx = jnp.max(scores, axis=1, keepdims=True)
    preds_ref[:] = jnp.min(jnp.where(scores == mx, it, NUM_CLASSES),
                           axis=1, keepdims=True)

    itc = lax.broadcasted_iota(jnp.int32, G.shape, 0)
    mxc = jnp.max(G, axis=0, keepdims=True)
    clusters_ref[:] = jnp.min(jnp.where(G == mxc, itc, C),
                              axis=0, keepdims=True)


def kernel(data, mu, S_inv, n, cluster_labels):
    xT = data.T
    mu3 = mu.reshape(C, D, 1)
    n2 = n.reshape(C, 1)
    scores, preds, clusters = pl.pallas_call(
        _egaussp_kernel,
        out_shape=[
            jax.ShapeDtypeStruct((B, NUM_CLASSES), jnp.float32),
            jax.ShapeDtypeStruct((B, 1), jnp.int32),
            jax.ShapeDtypeStruct((1, B), jnp.int32),
        ],
        scratch_shapes=[pltpu.VMEM((C, B), jnp.float32)],
    )(xT, mu3, S_inv, n2, cluster_labels)
    return (scores, preds[:, 0], clusters[0, :])


# speedup vs baseline: 1.1166x; 1.1166x over previous
import jax
import jax.numpy as jnp
from jax import lax
from jax.experimental import pallas as pl
from jax.experimental.pallas import tpu as pltpu

B = 1024
C = 512
D = 64
NUM_CLASSES = 10
KAPPA_N = 10.0
CB = 16


def _egaussp_kernel(xT_ref, mu3_ref, sinv_ref, n_ref, labels_ref,
                    scores_ref, preds_ref, clusters_ref, gamma_ref):
    xT = xT_ref[:]

    def step(i, carry):
        mu_b = mu3_ref[pl.ds(i * CB, CB), :, :]
        s_b = sinv_ref[pl.ds(i * CB, CB), :, :]
        diffT = xT[None, :, :] - mu_b
        tmpT = lax.dot_general(
            s_b, diffT, (((1,), (1,)), ((0,), (0,))),
            preferred_element_type=jnp.float32)
        d2 = jnp.sum(tmpT * diffT, axis=1)
        g = jnp.exp(-0.5 * d2)
        mask = n_ref[pl.ds(i * CB, CB), :] >= KAPPA_N
        g = jnp.where(mask, g, 0.0)
        gamma_ref[pl.ds(i * CB, CB), :] = g
        return carry

    lax.fori_loop(0, C // CB, step, 0, unroll=4)

    G = gamma_ref[:]
    denom = jnp.sum(G, axis=0) + 1e-12
    raw = lax.dot_general(
        G, labels_ref[:], (((0,), (0,)), ((), ())),
        preferred_element_type=jnp.float32)
    scores = raw / denom[:, None]
    scores_ref[:] = scores

    it = lax.broadcasted_iota(jnp.int32, scores.shape, 1)
    mx = jnp.max(scores, axis=1, keepdims=True)
    preds_ref[:] = jnp.min(jnp.where(scores == mx, it, NUM_CLASSES),
                           axis=1, keepdims=True)

    itc = lax.broadcasted_iota(jnp.int32, G.shape, 0)
    mxc = jnp.max(G, axis=0, keepdims=True)
    clusters_ref[:] = jnp.min(jnp.where(G == mxc, itc, C),
                              axis=0, keepdims=True)


def kernel(data, mu, S_inv, n, cluster_labels):
    xT = data.T
    mu3 = mu.reshape(C, D, 1)
    n2 = n.reshape(C, 1)
    scores, preds, clusters = pl.pallas_call(
        _egaussp_kernel,
        out_shape=[
            jax.ShapeDtypeStruct((B, NUM_CLASSES), jnp.float32),
            jax.ShapeDtypeStruct((B, 1), jnp.int32),
            jax.ShapeDtypeStruct((1, B), jnp.int32),
        ],
        scratch_shapes=[pltpu.VMEM((C, B), jnp.float32)],
    )(xT, mu3, S_inv, n2, cluster_labels)
    return (scores, preds[:, 0], clusters[0, :])


# speedup vs baseline: 1.1175x; 1.0008x over previous
import jax
import jax.numpy as jnp
from jax import lax
from jax.experimental import pallas as pl
from jax.experimental.pallas import tpu as pltpu

B = 1024
C = 512
D = 64
NUM_CLASSES = 10
KAPPA_N = 10.0
CB = 16


def _egaussp_kernel(xT_ref, mu3_ref, sinv_ref, n_ref, labels_ref,
                    scores_ref, preds_ref, clusters_ref, gamma_ref):
    xT = xT_ref[:]

    def step(i, carry):
        mu_b = mu3_ref[pl.ds(i * CB, CB), :, :]
        s_b = sinv_ref[pl.ds(i * CB, CB), :, :]
        diffT = xT[None, :, :] - mu_b
        tmpT = lax.dot_general(
            s_b, diffT, (((1,), (1,)), ((0,), (0,))),
            preferred_element_type=jnp.float32)
        acc = tmpT[:, 0:8, :] * diffT[:, 0:8, :]
        for j in range(1, D // 8):
            acc = acc + tmpT[:, 8 * j:8 * j + 8, :] * diffT[:, 8 * j:8 * j + 8, :]
        d2 = jnp.sum(acc, axis=1)
        g = jnp.exp(-0.5 * d2)
        mask = n_ref[pl.ds(i * CB, CB), :] >= KAPPA_N
        g = jnp.where(mask, g, 0.0)
        gamma_ref[pl.ds(i * CB, CB), :] = g
        return carry

    lax.fori_loop(0, C // CB, step, 0, unroll=4)

    G = gamma_ref[:]
    denom = jnp.sum(G, axis=0) + 1e-12
    raw = lax.dot_general(
        G, labels_ref[:], (((0,), (0,)), ((), ())),
        preferred_element_type=jnp.float32)
    scores = raw / denom[:, None]
    scores_ref[:] = scores

    it = lax.broadcasted_iota(jnp.int32, scores.shape, 1)
    mx = jnp.max(scores, axis=1, keepdims=True)
    preds_ref[:] = jnp.min(jnp.where(scores == mx, it, NUM_CLASSES),
                           axis=1, keepdims=True)

    itc = lax.broadcasted_iota(jnp.int32, G.shape, 0)
    mxc = jnp.max(G, axis=0, keepdims=True)
    clusters_ref[:] = jnp.min(jnp.where(G == mxc, itc, C),
                              axis=0, keepdims=True)


def kernel(data, mu, S_inv, n, cluster_labels):
    xT = data.T
    mu3 = mu.reshape(C, D, 1)
    n2 = n.reshape(C, 1)
    scores, preds, clusters = pl.pallas_call(
        _egaussp_kernel,
        out_shape=[
            jax.ShapeDtypeStruct((B, NUM_CLASSES), jnp.float32),
            jax.ShapeDtypeStruct((B, 1), jnp.int32),
            jax.ShapeDtypeStruct((1, B), jnp.int32),
        ],
        scratch_shapes=[pltpu.VMEM((C, B), jnp.float32)],
    )(xT, mu3, S_inv, n2, cluster_labels)
    return (scores, preds[:, 0], clusters[0, :])
